# Initial kernel scaffold; baseline (speedup 1.0000x reference)
import jax
import jax.numpy as jnp
from jax.experimental import pallas as pl


def kernel(x_flat, conv0_w, conv0_b, conv1_w, conv1_b, fc0_w, fc0_b, last_w, last_b, reg_w, reg_b):
    raise NotImplementedError("write your pallas kernel here")


# speedup vs baseline: 464.2332x; 464.2332x over previous
import functools

import jax
import jax.numpy as jnp
import numpy as np
from jax.experimental import pallas as pl
from jax.experimental.pallas import tpu as pltpu


def _sel(dj_w_j):
    ndj, nw, nj = dj_w_j
    t = np.zeros((ndj, nw, nj), np.float32)
    for dj in range(ndj):
        for j in range(nj):
            w = 2 * j - 1 + dj
            if 0 <= w < nw:
                t[dj, w, j] = 1.0
    return t


_T1 = _sel((3, 32, 16))
_T2 = _sel((3, 16, 8))


def _fused_kernel(bt, x_ref, a_ref, b_ref, b1_ref, b2_ref, fc_ref, fcb_ref,
                  lw_ref, lb_ref, rg_ref, rgb_ref, out_last_ref, out_reg_ref):
    f32 = jnp.float32
    bf16 = jnp.bfloat16
    dot = functools.partial(jnp.dot, preferred_element_type=f32)

    x0 = x_ref[0].reshape(8 * bt, 128)
    x1 = x_ref[1].reshape(8 * bt, 128)
    x2 = x_ref[2].reshape(8 * bt, 128)
    x3 = x_ref[3].reshape(8 * bt, 128)

    a0, a1, a2 = a_ref[0], a_ref[1], a_ref[2]
    b1 = b1_ref[...]

    zx = jnp.zeros((bt, 128), bf16)
    x3s = jnp.concatenate([zx, x3[: 7 * bt]], axis=0)
    h_e = jnp.maximum(dot(x3s, a0) + dot(x0, a1) + dot(x1, a2) + b1, 0.0)
    h_o = jnp.maximum(dot(x1, a0) + dot(x2, a1) + dot(x3, a2) + b1, 0.0)
    h_e = h_e.astype(bf16)
    h_o = h_o.astype(bf16)

    zh = jnp.zeros((bt, 256), bf16)
    h_os = jnp.concatenate([zh, h_o[: 7 * bt]], axis=0)
    out2 = jnp.maximum(
        dot(h_os, b_ref[0]) + dot(h_e, b_ref[1]) + dot(h_o, b_ref[2])
        + b2_ref[...], 0.0).astype(bf16)

    hacc = dot(out2[:bt], fc_ref[0])
    racc = dot(out2[:bt], rg_ref[0])
    for i2 in range(1, 8):
        blk = out2[i2 * bt:(i2 + 1) * bt]
        hacc += dot(blk, fc_ref[i2])
        racc += dot(blk, rg_ref[i2])

    h = jnp.maximum(hacc + fcb_ref[...], 0.0).astype(bf16)
    out_last_ref[...] = dot(h, lw_ref[...]) + lb_ref[...]
    out_reg_ref[...] = racc + rgb_ref[...]


def kernel(x_flat, conv0_w, conv0_b, conv1_w, conv1_b, fc0_w, fc0_b,
           last_w, last_b, reg_w, reg_b):
    f32 = jnp.float32
    bf16 = jnp.bfloat16
    B = x_flat.shape[0]
    bt = 128 if B % 128 == 0 else B

    xr = x_flat.reshape(B, 4, 8, 4, 32).transpose(3, 2, 0, 4, 1)
    xr = xr.reshape(4, 8, B, 128).astype(bf16)

    amat = jnp.einsum("dwj,ocid->iwcjo", _T1, conv0_w).reshape(3, 128, 256)
    bmat = jnp.einsum("dab,ocid->iacbo", _T2, conv1_w).reshape(3, 256, 256)
    b1row = jnp.tile(conv0_b, 16).reshape(1, 256)
    b2row = jnp.tile(conv1_b, 8).reshape(1, 256)

    fcr = fc0_w.reshape(32, 8, 8, 256).transpose(1, 2, 0, 3).reshape(8, 256, 256)
    rgr = reg_w.reshape(32, 8, 8, 64).transpose(1, 2, 0, 3).reshape(8, 256, 64)

    full = lambda a: pl.BlockSpec(a.shape, lambda i: (0,) * a.ndim)
    weights = [amat.astype(bf16), bmat.astype(bf16), b1row, b2row,
               fcr.astype(bf16), fc0_b.reshape(1, 256).astype(f32),
               last_w.astype(bf16), last_b.reshape(1, 128).astype(f32),
               rgr.astype(bf16), reg_b.reshape(1, 64).astype(f32)]

    out_last, out_reg = pl.pallas_call(
        functools.partial(_fused_kernel, bt),
        out_shape=(jax.ShapeDtypeStruct((B, 128), f32),
                   jax.ShapeDtypeStruct((B, 64), f32)),
        grid=(B // bt,),
        in_specs=[pl.BlockSpec((4, 8, bt, 128), lambda i: (0, 0, i, 0))]
        + [full(w) for w in weights],
        out_specs=[pl.BlockSpec((bt, 128), lambda i: (i, 0)),
                   pl.BlockSpec((bt, 64), lambda i: (i, 0))],
        compiler_params=pltpu.CompilerParams(
            dimension_semantics=("parallel",)),
    )(xr, *weights)
    return out_last, out_reg


# speedup vs baseline: 1012.9035x; 2.1819x over previous
import functools

import jax
import jax.numpy as jnp
import numpy as np
from jax.experimental import pallas as pl
from jax.experimental.pallas import tpu as pltpu


def _sel(ndj, nw, nj):
    t = np.zeros((ndj, nw, nj), np.float32)
    for dj in range(ndj):
        for j in range(nj):
            w = 2 * j - 1 + dj
            if 0 <= w < nw:
                t[dj, w, j] = 1.0
    return t


_T1 = _sel(3, 32, 16)
_T2 = _sel(3, 16, 8)


def _fused_kernel(bt, x_ref, a_ref, b_ref, b1_ref, b2_ref, fc_ref, fcb_ref,
                  lw_ref, lb_ref, rg_ref, rgb_ref, out_last_ref, out_reg_ref):
    f32 = jnp.float32
    bf16 = jnp.bfloat16
    dot = functools.partial(jnp.dot, preferred_element_type=f32)

    xb = x_ref[...].astype(bf16)
    xp = []
    for p in range(4):
        rows = []
        for i2 in range(8):
            h = 4 * i2 + p
            rows.append(jnp.concatenate(
                [xb[:, c * 1024 + h * 32: c * 1024 + h * 32 + 32]
                 for c in range(4)], axis=1))
        xp.append(jnp.concatenate(rows, axis=0))
    x0, x1, x2, x3 = xp

    a0, a1, a2 = a_ref[0], a_ref[1], a_ref[2]
    b1 = b1_ref[...]

    zx = jnp.zeros((bt, 128), bf16)
    x3s = jnp.concatenate([zx, x3[: 7 * bt]], axis=0)
    h_e = jnp.maximum(dot(x3s, a0) + dot(x0, a1) + dot(x1, a2) + b1, 0.0)
    h_o = jnp.maximum(dot(x1, a0) + dot(x2, a1) + dot(x3, a2) + b1, 0.0)
    h_e = h_e.astype(bf16)
    h_o = h_o.astype(bf16)

    zh = jnp.zeros((bt, 256), bf16)
    h_os = jnp.concatenate([zh, h_o[: 7 * bt]], axis=0)
    out2 = jnp.maximum(
        dot(h_os, b_ref[0]) + dot(h_e, b_ref[1]) + dot(h_o, b_ref[2])
        + b2_ref[...], 0.0).astype(bf16)

    hacc = dot(out2[:bt], fc_ref[0])
    racc = dot(out2[:bt], rg_ref[0])
    for i2 in range(1, 8):
        blk = out2[i2 * bt:(i2 + 1) * bt]
        hacc += dot(blk, fc_ref[i2])
        racc += dot(blk, rg_ref[i2])

    h = jnp.maximum(hacc + fcb_ref[...], 0.0).astype(bf16)
    out_last_ref[...] = dot(h, lw_ref[...]) + lb_ref[...]
    out_reg_ref[...] = racc + rgb_ref[...]


def kernel(x_flat, conv0_w, conv0_b, conv1_w, conv1_b, fc0_w, fc0_b,
           last_w, last_b, reg_w, reg_b):
    f32 = jnp.float32
    bf16 = jnp.bfloat16
    B = x_flat.shape[0]
    bt = 128 if B % 128 == 0 else B

    amat = jnp.einsum("dwj,ocid->icwjo", _T1, conv0_w).reshape(3, 128, 256)
    bmat = jnp.einsum("dab,ocid->iacbo", _T2, conv1_w).reshape(3, 256, 256)
    b1row = jnp.tile(conv0_b, 16).reshape(1, 256)
    b2row = jnp.tile(conv1_b, 8).reshape(1, 256)

    fcr = fc0_w.reshape(32, 8, 8, 256).transpose(1, 2, 0, 3).reshape(8, 256, 256)
    rgr = reg_w.reshape(32, 8, 8, 64).transpose(1, 2, 0, 3).reshape(8, 256, 64)

    full = lambda a: pl.BlockSpec(a.shape, lambda i: (0,) * a.ndim)
    weights = [amat.astype(bf16), bmat.astype(bf16), b1row, b2row,
               fcr.astype(bf16), fc0_b.reshape(1, 256).astype(f32),
               last_w.astype(bf16), last_b.reshape(1, 128).astype(f32),
               rgr.astype(bf16), reg_b.reshape(1, 64).astype(f32)]

    out_last, out_reg = pl.pallas_call(
        functools.partial(_fused_kernel, bt),
        out_shape=(jax.ShapeDtypeStruct((B, 128), f32),
                   jax.ShapeDtypeStruct((B, 64), f32)),
        grid=(B // bt,),
        in_specs=[pl.BlockSpec((bt, 4096), lambda i: (i, 0))]
        + [full(w) for w in weights],
        out_specs=[pl.BlockSpec((bt, 128), lambda i: (i, 0)),
                   pl.BlockSpec((bt, 64), lambda i: (i, 0))],
        compiler_params=pltpu.CompilerParams(
            dimension_semantics=("parallel",)),
    )(x_flat, *weights)
    return out_last, out_reg


# speedup vs baseline: 1052.4336x; 1.0390x over previous
import functools

import jax
import jax.numpy as jnp
import numpy as np
from jax.experimental import pallas as pl
from jax.experimental.pallas import tpu as pltpu


def _sel(ndj, nw, nj):
    t = np.zeros((ndj, nw, nj), np.float32)
    for dj in range(ndj):
        for j in range(nj):
            w = 2 * j - 1 + dj
            if 0 <= w < nw:
                t[dj, w, j] = 1.0
    return t


_T1 = _sel(3, 32, 16)
_T2 = _sel(3, 16, 8)


def _fused_kernel(bt, x_ref, a_ref, b_ref, b1_ref, b2_ref, fc_ref, fcb_ref,
                  lw_ref, lb_ref, rg_ref, rgb_ref, out_last_ref, out_reg_ref):
    f32 = jnp.float32
    bf16 = jnp.bfloat16
    dot = functools.partial(jnp.dot, preferred_element_type=f32)

    xb = x_ref[...].astype(bf16)
    xp = []
    for p in range(4):
        rows = []
        for i2 in range(8):
            h = 4 * i2 + p
            rows.append(jnp.concatenate(
                [xb[:, c * 1024 + h * 32: c * 1024 + h * 32 + 32]
                 for c in range(4)], axis=1))
        xp.append(jnp.concatenate(rows, axis=0))
    x0, x1, x2, x3 = xp

    a0, a1, a2 = a_ref[0], a_ref[1], a_ref[2]
    b1 = b1_ref[...]

    zx = jnp.zeros((bt, 128), bf16)
    x3s = jnp.concatenate([zx, x3[: 7 * bt]], axis=0)
    h_e = jnp.maximum(dot(x3s, a0) + dot(x0, a1) + dot(x1, a2) + b1, 0.0)
    h_o = jnp.maximum(dot(x1, a0) + dot(x2, a1) + dot(x3, a2) + b1, 0.0)
    h_e = h_e.astype(bf16)
    h_o = h_o.astype(bf16)

    zh = jnp.zeros((bt, 256), bf16)
    h_os = jnp.concatenate([zh, h_o[: 7 * bt]], axis=0)
    out2 = jnp.maximum(
        dot(h_os, b_ref[0]) + dot(h_e, b_ref[1]) + dot(h_o, b_ref[2])
        + b2_ref[...], 0.0).astype(bf16)

    hacc = dot(out2[:bt], fc_ref[0])
    racc = dot(out2[:bt], rg_ref[0])
    for i2 in range(1, 8):
        blk = out2[i2 * bt:(i2 + 1) * bt]
        hacc += dot(blk, fc_ref[i2])
        racc += dot(blk, rg_ref[i2])

    h = jnp.maximum(hacc + fcb_ref[...], 0.0).astype(bf16)
    out_last_ref[...] = dot(h, lw_ref[...]) + lb_ref[...]
    out_reg_ref[...] = racc + rgb_ref[...]


def kernel(x_flat, conv0_w, conv0_b, conv1_w, conv1_b, fc0_w, fc0_b,
           last_w, last_b, reg_w, reg_b):
    f32 = jnp.float32
    bf16 = jnp.bfloat16
    B = x_flat.shape[0]
    bt = 256 if B % 256 == 0 else B

    amat = jnp.einsum("dwj,ocid->icwjo", _T1, conv0_w).reshape(3, 128, 256)
    bmat = jnp.einsum("dab,ocid->iacbo", _T2, conv1_w).reshape(3, 256, 256)
    b1row = jnp.tile(conv0_b, 16).reshape(1, 256)
    b2row = jnp.tile(conv1_b, 8).reshape(1, 256)

    fcr = fc0_w.reshape(32, 8, 8, 256).transpose(1, 2, 0, 3).reshape(8, 256, 256)
    rgr = reg_w.reshape(32, 8, 8, 64).transpose(1, 2, 0, 3).reshape(8, 256, 64)

    full = lambda a: pl.BlockSpec(a.shape, lambda i: (0,) * a.ndim)
    weights = [amat.astype(bf16), bmat.astype(bf16), b1row, b2row,
               fcr.astype(bf16), fc0_b.reshape(1, 256).astype(f32),
               last_w.astype(bf16), last_b.reshape(1, 128).astype(f32),
               rgr.astype(bf16), reg_b.reshape(1, 64).astype(f32)]

    out_last, out_reg = pl.pallas_call(
        functools.partial(_fused_kernel, bt),
        out_shape=(jax.ShapeDtypeStruct((B, 128), f32),
                   jax.ShapeDtypeStruct((B, 64), f32)),
        grid=(B // bt,),
        in_specs=[pl.BlockSpec((bt, 4096), lambda i: (i, 0))]
        + [full(w) for w in weights],
        out_specs=[pl.BlockSpec((bt, 128), lambda i: (i, 0)),
                   pl.BlockSpec((bt, 64), lambda i: (i, 0))],
        compiler_params=pltpu.CompilerParams(
            dimension_semantics=("parallel",)),
    )(x_flat, *weights)
    return out_last, out_reg
